# baseline (speedup 1.0000x reference)
import functools

import jax
import jax.numpy as jnp
from jax import lax
from jax.experimental import pallas as pl
from jax.experimental.pallas import tpu as pltpu
from jax.experimental.pallas import tpu_sc as plsc

NC = 2
NS = 16
NW = NC * NS

B = 16384
D = 64
K = 20
J = K + 1
BPW = B // NW
C = 32
NCH = BPW // C
SCOL = 32
VROWS = C * J


def _sc_body(gt_hbm, gidx_hbm, pu_hbm, pv_hbm, u_hbm, v_hbm, out_hbm,
             gt_v, gidx_v, pu_v, pv_v, urows_v, vrows_v, scores_v, sem):
    wid = lax.axis_index("s") * NC + lax.axis_index("c")
    base = wid * BPW
    pltpu.sync_copy(gt_hbm.at[pl.ds(base, BPW)], gt_v)
    pltpu.sync_copy(pu_hbm.at[pl.ds(base, BPW)], pu_v)
    pltpu.sync_copy(gidx_hbm.at[pl.ds(base * J, BPW * J)], gidx_v)
    pltpu.sync_copy(pv_hbm.at[pl.ds(base * J, BPW * J)], pv_v)

    for c in range(NCH):
        cps = [pltpu.async_copy(u_hbm.at[gt_v.at[pl.ds(c * C, C)]],
                                urows_v, sem)]
        roff = c * VROWS
        nfull, tail = VROWS // 128, VROWS % 128
        for i in range(nfull):
            cps.append(pltpu.async_copy(
                v_hbm.at[gidx_v.at[pl.ds(roff + i * 128, 128)]],
                vrows_v.at[pl.ds(i * 128, 128)], sem))
        if tail:
            cps.append(pltpu.async_copy(
                v_hbm.at[gidx_v.at[pl.ds(roff + nfull * 128, tail)]],
                vrows_v.at[pl.ds(nfull * 128, tail)], sem))
        for cp in cps:
            cp.wait()

        lanes = lax.iota(jnp.int32, 16)
        m15 = lanes == 15
        zero = lanes * 0

        def bbody(b, carry):
            pu_b = plsc.load_gather(pu_v, [zero + (c * C + b)])
            u = [jnp.where(pu_b == 1,
                           urows_v[b, pl.ds(64 + 16 * q, 16)],
                           urows_v[b, pl.ds(16 * q, 16)])
                 for q in range(4)]
            r0 = b * J
            bfull = zero + b
            f0 = zero + (c * C + b) * J
            for j in range(J):
                pv_bj = plsc.load_gather(pv_v, [f0 + j])
                p = zero.astype(jnp.float32)
                for q in range(4):
                    rsel = jnp.where(pv_bj == 1,
                                     vrows_v[r0 + j, pl.ds(64 + 16 * q, 16)],
                                     vrows_v[r0 + j, pl.ds(16 * q, 16)])
                    p = p + u[q] * rsel
                cs = plsc.cumsum(p)
                plsc.store_scatter(scores_v, [bfull, zero + j], cs,
                                   mask=m15)
            return carry

        lax.fori_loop(0, C, bbody, 0)
        pltpu.sync_copy(scores_v, out_hbm.at[pl.ds(base + c * C, C), :])


_sc_scores = functools.partial(
    pl.kernel, _sc_body,
    out_type=jax.ShapeDtypeStruct((B, SCOL), jnp.float32),
    mesh=plsc.VectorSubcoreMesh(core_axis_name="c", subcore_axis_name="s",
                                num_cores=NC, num_subcores=NS),
    compiler_params=pltpu.CompilerParams(needs_layout_passes=False),
    scratch_types=[
        pltpu.VMEM((BPW,), jnp.int32),
        pltpu.VMEM((BPW * J,), jnp.int32),
        pltpu.VMEM((BPW,), jnp.int32),
        pltpu.VMEM((BPW * J,), jnp.int32),
        pltpu.VMEM((C, 2 * D), jnp.float32),
        pltpu.VMEM((VROWS, 2 * D), jnp.float32),
        pltpu.VMEM((C, SCOL), jnp.float32),
        pltpu.SemaphoreType.DMA,
    ],
)()


def _tc_finish_body(s_ref, o_ref):
    x = s_ref[...]
    col = lax.broadcasted_iota(jnp.int32, x.shape, 1)
    y = jnp.where(col == 0, -x, x)
    sp = jnp.maximum(y, 0.0) + jnp.log1p(jnp.exp(-jnp.abs(y)))
    sp = jnp.where(col < J, sp, 0.0)
    o_ref[...] = jnp.sum(sp, axis=1)


_TCR = 2048


def _tc_finish(scores):
    return pl.pallas_call(
        _tc_finish_body,
        grid=(B // _TCR,),
        in_specs=[pl.BlockSpec((_TCR, SCOL), lambda i: (i, 0))],
        out_specs=pl.BlockSpec((_TCR,), lambda i: (i,)),
        out_shape=jax.ShapeDtypeStruct((B,), jnp.float32),
    )(scores)


def kernel(target, context, neg, u_weight, v_weight):
    tgt = target.astype(jnp.int32)
    cat = jnp.concatenate(
        [context.astype(jnp.int32)[:, None], neg.astype(jnp.int32)],
        axis=1).reshape(-1)
    u2 = u_weight.reshape(500000, 128)
    v2 = v_weight.reshape(500000, 128)
    scores = _sc_scores(tgt >> 1, cat >> 1, tgt & 1, cat & 1, u2, v2)
    return _tc_finish(scores)
